# baseline (speedup 1.0000x reference)
import functools

import jax
import jax.numpy as jnp
from jax import lax
from jax.experimental import pallas as pl
from jax.experimental.pallas import tpu as pltpu
from jax.experimental.pallas import tpu_sc as plsc

N_NODES = 10000
N_EDGES = 320000
NUM_RELS = 5
D = 128

N_TILES = 16
N_SC = 2
B = 96
MAIN_BATCHES = 108
E_PAD = N_SC * N_TILES * MAIN_BATCHES * B
NB_TOTAL = E_PAD // B
N_PAD_ROWS = 240
DEG_PAD_SLOTS = 176
N_ACC = N_NODES + N_PAD_ROWS
DEG_N = NUM_RELS * N_NODES + DEG_PAD_SLOTS
ROWS_PER_TILE = N_ACC // N_TILES
DEG_PER_TILE = DEG_N // N_TILES

BLK = 1000


def _transform_body(feat_ref, W0_ref, W1_ref, b_ref, Wrel_ref, T_ref):
    i = pl.program_id(0)
    first_half = i < (5000 // BLK)
    W = jnp.where(first_half, W0_ref[...], W1_ref[...])
    b = jnp.where(first_half, b_ref[0], b_ref[1])
    h = jnp.dot(feat_ref[...], W, preferred_element_type=jnp.float32) + b
    for rr in range(NUM_RELS):
        T_ref[rr] = jnp.dot(h, Wrel_ref[rr], preferred_element_type=jnp.float32)


def _finish_body(p_ref, brel_ref, o_ref):
    o_ref[...] = jnp.maximum(p_ref[0] + p_ref[1] + brel_ref[...], 0.0)


def _recip_body(p_ref, o_ref):
    d = p_ref[0] + p_ref[1]
    o_ref[...] = jnp.where(d > 0, 1.0 / jnp.maximum(d, 1.0), 0.0)


def _sc_count_body(pk_hbm, degp_hbm,
                   pk_a, pk_b, ones_v, zdeg_v, deg_sh, spk_a, spk_b):
    c = lax.axis_index("c")
    s = lax.axis_index("s")
    w = c * N_TILES + s

    pk = (pk_a, pk_b)
    spk = (spk_a, spk_b)

    one16 = jnp.ones((16,), jnp.float32)
    zero16 = jnp.zeros((16,), jnp.float32)
    for g in range(B // 16):
        ones_v[pl.ds(g * 16, 16)] = one16

    def zdeg_body(i, _):
        zdeg_v[pl.ds(i * 16, 16)] = zero16
        return 0

    lax.fori_loop(0, DEG_PER_TILE // 16, zdeg_body, 0)
    pltpu.sync_copy(zdeg_v, deg_sh.at[pl.ds(s * DEG_PER_TILE, DEG_PER_TILE)])
    plsc.subcore_barrier()

    def cnt_start_pack(k, st):
        pltpu.async_copy(pk_hbm.at[w * MAIN_BATCHES + k], pk[st], spk[st])

    def cnt_wait_pack(k, st):
        pltpu.make_async_copy(pk_hbm.at[w * MAIN_BATCHES + k], pk[st],
                              spk[st]).wait()

    def cnt_half(k, st):
        ot = 1 - st

        @pl.when(k + 1 < MAIN_BATCHES)
        def _():
            cnt_start_pack(k + 1, ot)

        cnt_wait_pack(k, st)
        pltpu.sync_copy(ones_v, deg_sh.at[pk[st].at[1]], add=True)

    cnt_start_pack(0, 0)

    def cnt_body(j, _):
        cnt_half(2 * j, 0)
        cnt_half(2 * j + 1, 1)
        return 0

    lax.fori_loop(0, MAIN_BATCHES // 2, cnt_body, 0)
    plsc.subcore_barrier()

    pltpu.sync_copy(deg_sh.at[pl.ds(s * DEG_PER_TILE, DEG_PER_TILE)], zdeg_v)
    pltpu.sync_copy(zdeg_v,
                    degp_hbm.at[pl.ds(c * DEG_N + s * DEG_PER_TILE,
                                      DEG_PER_TILE)])


_sc_count = functools.partial(
    pl.kernel,
    out_type=jax.ShapeDtypeStruct((N_SC * DEG_N,), jnp.float32),
    mesh=plsc.VectorSubcoreMesh(core_axis_name="c", subcore_axis_name="s"),
    scratch_types=[
        pltpu.VMEM((3, B), jnp.int32),
        pltpu.VMEM((3, B), jnp.int32),
        pltpu.VMEM((B,), jnp.float32),
        pltpu.VMEM((DEG_PER_TILE,), jnp.float32),
        pltpu.VMEM_SHARED((DEG_N,), jnp.float32),
        pltpu.SemaphoreType.DMA,
        pltpu.SemaphoreType.DMA,
    ],
)(_sc_count_body)


def _sc_agg_body(T_hbm, pk_hbm, rec_hbm, out_hbm,
                 pk_0, pk_1, pk_2, rows_0, rows_1, rows_2,
                 rec_0, rec_1, rec_2, acc_sh,
                 spk_0, spk_1, spk_2, srow_0, srow_1, srow_2,
                 sdeg_0, sdeg_1, sdeg_2, ssc_0, ssc_1, ssc_2):
    c = lax.axis_index("c")
    s = lax.axis_index("s")

    pk = (pk_0, pk_1, pk_2)
    rows = (rows_0, rows_1, rows_2)
    rec = (rec_0, rec_1, rec_2)
    spk = (spk_0, spk_1, spk_2)
    srow = (srow_0, srow_1, srow_2)
    sdeg = (sdeg_0, sdeg_1, sdeg_2)
    ssc = (ssc_0, ssc_1, ssc_2)

    zero16 = jnp.zeros((16,), jnp.float32)

    def zrow_body(i, _):
        for c8 in range(D // 16):
            rows_0[i, pl.ds(c8 * 16, 16)] = zero16
        return 0

    lax.fori_loop(0, B, zrow_body, 0)

    for rep in range(ROWS_PER_TILE // B):
        pltpu.sync_copy(rows_0,
                        acc_sh.at[pl.ds(s * ROWS_PER_TILE + rep * B, B)])
    _tail = ROWS_PER_TILE - (ROWS_PER_TILE // B) * B
    if _tail:
        pltpu.sync_copy(
            rows_0.at[pl.ds(0, _tail)],
            acc_sh.at[pl.ds(s * ROWS_PER_TILE + ROWS_PER_TILE - _tail,
                            _tail)])
    plsc.subcore_barrier()

    w = c * N_TILES + s

    dnums = lax.GatherDimensionNumbers(
        offset_dims=(), collapsed_slice_dims=(0,), start_index_map=(0,))

    def m_start_pack(i, st):
        pltpu.async_copy(pk_hbm.at[w * MAIN_BATCHES + i], pk[st], spk[st])

    def m_wait_pack(i, st):
        pltpu.make_async_copy(pk_hbm.at[w * MAIN_BATCHES + i], pk[st],
                              spk[st]).wait()

    def m_start_gathers(st):
        pltpu.async_copy(T_hbm.at[pk[st].at[0]], rows[st], srow[st])
        pltpu.async_copy(rec_hbm.at[pk[st].at[1]], rec[st], sdeg[st])

    def m_wait_gathers(st):
        pltpu.make_async_copy(T_hbm.at[pk[st].at[0]], rows[st],
                              srow[st]).wait()
        pltpu.make_async_copy(rec_hbm.at[pk[st].at[1]], rec[st],
                              sdeg[st]).wait()

    def m_wait_scatter(st):
        pltpu.make_async_copy(rows[st], acc_sh.at[pk[st].at[2]],
                              ssc[st]).wait()

    def m_scale(st):
        rows_st, rec_st = rows[st], rec[st]

        def grp(g, _):
            r16 = rec_st[pl.ds(g * 16, 16)]
            for l in range(16):
                bc = lax.gather(r16, jnp.full((16, 1), l, jnp.int32),
                                dnums, slice_sizes=(1,),
                                mode=lax.GatherScatterMode.PROMISE_IN_BOUNDS)
                row = g * 16 + l
                for c8 in range(D // 16):
                    sl = pl.ds(c8 * 16, 16)
                    rows_st[row, sl] = rows_st[row, sl] * bc
            return 0

        lax.fori_loop(0, B // 16, grp, 0)

    def m_half(i, st, first=False, last=False):
        nxt = (st + 1) % 3
        if not first:
            m_wait_scatter(nxt)
        if not last:
            m_start_pack(i + 1, nxt)
        m_wait_gathers(st)
        if not last:
            m_wait_pack(i + 1, nxt)
            m_start_gathers(nxt)
        m_scale(st)
        pltpu.async_copy(rows[st], acc_sh.at[pk[st].at[2]], ssc[st],
                         add=True)

    m_start_pack(0, 0)
    m_wait_pack(0, 0)
    m_start_gathers(0)
    m_half(0, 0, first=True)
    m_half(1, 1, first=True)

    def m_body(j, _):
        i = 3 * j + 2
        m_half(i, 2)
        m_half(i + 1, 0)
        m_half(i + 2, 1)
        return 0

    lax.fori_loop(0, (MAIN_BATCHES - 3) // 3, m_body, 0)
    m_half(MAIN_BATCHES - 1, (MAIN_BATCHES - 1) % 3, last=True)
    m_wait_scatter((MAIN_BATCHES - 2) % 3)
    m_wait_scatter((MAIN_BATCHES - 1) % 3)
    plsc.subcore_barrier()

    pltpu.sync_copy(acc_sh.at[pl.ds(s * ROWS_PER_TILE, ROWS_PER_TILE)],
                    out_hbm.at[c, pl.ds(s * ROWS_PER_TILE, ROWS_PER_TILE)])


_sc_agg = functools.partial(
    pl.kernel,
    out_type=jax.ShapeDtypeStruct((N_SC, N_ACC, D), jnp.float32),
    mesh=plsc.VectorSubcoreMesh(core_axis_name="c", subcore_axis_name="s"),
    scratch_types=[
        pltpu.VMEM((3, B), jnp.int32),
        pltpu.VMEM((3, B), jnp.int32),
        pltpu.VMEM((3, B), jnp.int32),
        pltpu.VMEM((B, D), jnp.float32),
        pltpu.VMEM((B, D), jnp.float32),
        pltpu.VMEM((B, D), jnp.float32),
        pltpu.VMEM((B,), jnp.float32),
        pltpu.VMEM((B,), jnp.float32),
        pltpu.VMEM((B,), jnp.float32),
        pltpu.VMEM_SHARED((N_ACC, D), jnp.float32),
        pltpu.SemaphoreType.DMA,
        pltpu.SemaphoreType.DMA,
        pltpu.SemaphoreType.DMA,
        pltpu.SemaphoreType.DMA,
        pltpu.SemaphoreType.DMA,
        pltpu.SemaphoreType.DMA,
        pltpu.SemaphoreType.DMA,
        pltpu.SemaphoreType.DMA,
        pltpu.SemaphoreType.DMA,
        pltpu.SemaphoreType.DMA,
        pltpu.SemaphoreType.DMA,
        pltpu.SemaphoreType.DMA,
    ],
)(_sc_agg_body)


def kernel(feat0, feat1, edge_index, r, W0, b0, W1, b1, Wrel, brel):
    feat = jnp.concatenate([feat0, feat1], axis=0)
    bstack = jnp.stack([b0, b1])[:, None, :]

    T = pl.pallas_call(
        _transform_body,
        grid=(N_NODES // BLK,),
        in_specs=[
            pl.BlockSpec((BLK, D), lambda i: (i, 0)),
            pl.BlockSpec((D, D), lambda i: (0, 0)),
            pl.BlockSpec((D, D), lambda i: (0, 0)),
            pl.BlockSpec((2, 1, D), lambda i: (0, 0, 0)),
            pl.BlockSpec((NUM_RELS, D, D), lambda i: (0, 0, 0)),
        ],
        out_specs=pl.BlockSpec((NUM_RELS, BLK, D), lambda i: (0, i, 0)),
        out_shape=jax.ShapeDtypeStruct((NUM_RELS, N_NODES, D), jnp.float32),
    )(feat, W0, W1, bstack, Wrel)
    T_flat = T.reshape(NUM_RELS * N_NODES, D)

    u, v = edge_index[0], edge_index[1]
    key_ru = r * N_NODES + u
    key_rv = r * N_NODES + v
    npad = E_PAD - N_EDGES
    pi = jnp.arange(npad, dtype=jnp.int32)
    ru_p = jnp.concatenate([key_ru, pi % 512])
    rv_p = jnp.concatenate([key_rv, NUM_RELS * N_NODES + pi % DEG_PAD_SLOTS])
    vd_p = jnp.concatenate([v, N_NODES + pi % N_PAD_ROWS])
    pack = jnp.stack([ru_p.reshape(NB_TOTAL, B), rv_p.reshape(NB_TOTAL, B),
                      vd_p.reshape(NB_TOTAL, B)], axis=1)

    degp = _sc_count(pack)
    rec_table = pl.pallas_call(
        _recip_body,
        grid=(1,),
        in_specs=[pl.BlockSpec((N_SC, DEG_N // D, D), lambda i: (0, 0, 0))],
        out_specs=pl.BlockSpec((DEG_N // D, D), lambda i: (0, 0)),
        out_shape=jax.ShapeDtypeStruct((DEG_N // D, D), jnp.float32),
    )(degp.reshape(N_SC, DEG_N // D, D)).reshape(DEG_N)

    partial = _sc_agg(T_flat, pack, rec_table)

    out = pl.pallas_call(
        _finish_body,
        grid=(N_NODES // BLK,),
        in_specs=[
            pl.BlockSpec((N_SC, BLK, D), lambda i: (0, i, 0)),
            pl.BlockSpec((1, D), lambda i: (0, 0)),
        ],
        out_specs=pl.BlockSpec((BLK, D), lambda i: (i, 0)),
        out_shape=jax.ShapeDtypeStruct((N_NODES, D), jnp.float32),
    )(partial, brel[None, :])
    return out


# speedup vs baseline: 1.0449x; 1.0449x over previous
import functools

import jax
import jax.numpy as jnp
from jax import lax
from jax.experimental import pallas as pl
from jax.experimental.pallas import tpu as pltpu
from jax.experimental.pallas import tpu_sc as plsc

N_NODES = 10000
N_EDGES = 320000
NUM_RELS = 5
D = 128

N_TILES = 16
N_SC = 2
B = 112
MAIN_BATCHES = 93
E_PAD = N_SC * N_TILES * MAIN_BATCHES * B
NB_TOTAL = E_PAD // B
N_PAD_ROWS = 112
DEG_PAD_SLOTS = 176
N_ACC = N_NODES + N_PAD_ROWS
DEG_N = NUM_RELS * N_NODES + DEG_PAD_SLOTS
ROWS_PER_TILE = N_ACC // N_TILES
DEG_PER_TILE = DEG_N // N_TILES

BLK = 1000


def _transform_body(feat_ref, W0_ref, W1_ref, b_ref, Wrel_ref, T_ref):
    i = pl.program_id(0)
    first_half = i < (5000 // BLK)
    W = jnp.where(first_half, W0_ref[...], W1_ref[...])
    b = jnp.where(first_half, b_ref[0], b_ref[1])
    h = jnp.dot(feat_ref[...], W, preferred_element_type=jnp.float32) + b
    for rr in range(NUM_RELS):
        T_ref[rr] = jnp.dot(h, Wrel_ref[rr], preferred_element_type=jnp.float32)


def _finish_body(p_ref, brel_ref, o_ref):
    o_ref[...] = jnp.maximum(p_ref[0] + p_ref[1] + brel_ref[...], 0.0)


def _recip_body(p_ref, o_ref):
    d = p_ref[0] + p_ref[1]
    o_ref[...] = jnp.where(d > 0, 1.0 / jnp.maximum(d, 1.0), 0.0)


def _sc_count_body(pk_hbm, degp_hbm,
                   pk_a, pk_b, ones_v, zdeg_v, deg_sh, spk_a, spk_b):
    c = lax.axis_index("c")
    s = lax.axis_index("s")
    w = c * N_TILES + s

    pk = (pk_a, pk_b)
    spk = (spk_a, spk_b)

    one16 = jnp.ones((16,), jnp.float32)
    zero16 = jnp.zeros((16,), jnp.float32)
    for g in range(B // 16):
        ones_v[pl.ds(g * 16, 16)] = one16

    def zdeg_body(i, _):
        zdeg_v[pl.ds(i * 16, 16)] = zero16
        return 0

    lax.fori_loop(0, DEG_PER_TILE // 16, zdeg_body, 0)
    pltpu.sync_copy(zdeg_v, deg_sh.at[pl.ds(s * DEG_PER_TILE, DEG_PER_TILE)])
    plsc.subcore_barrier()

    def cnt_start_pack(k, st):
        pltpu.async_copy(pk_hbm.at[w * MAIN_BATCHES + k], pk[st], spk[st])

    def cnt_wait_pack(k, st):
        pltpu.make_async_copy(pk_hbm.at[w * MAIN_BATCHES + k], pk[st],
                              spk[st]).wait()

    def cnt_half(k, st):
        ot = 1 - st

        @pl.when(k + 1 < MAIN_BATCHES)
        def _():
            cnt_start_pack(k + 1, ot)

        cnt_wait_pack(k, st)
        pltpu.sync_copy(ones_v, deg_sh.at[pk[st].at[1]], add=True)

    cnt_start_pack(0, 0)

    def cnt_body(j, _):
        cnt_half(2 * j, 0)
        cnt_half(2 * j + 1, 1)
        return 0

    lax.fori_loop(0, MAIN_BATCHES // 2, cnt_body, 0)
    cnt_wait_pack(MAIN_BATCHES - 1, 0)
    pltpu.sync_copy(ones_v, deg_sh.at[pk[0].at[1]], add=True)
    plsc.subcore_barrier()

    pltpu.sync_copy(deg_sh.at[pl.ds(s * DEG_PER_TILE, DEG_PER_TILE)], zdeg_v)
    pltpu.sync_copy(zdeg_v,
                    degp_hbm.at[pl.ds(c * DEG_N + s * DEG_PER_TILE,
                                      DEG_PER_TILE)])


_sc_count = functools.partial(
    pl.kernel,
    out_type=jax.ShapeDtypeStruct((N_SC * DEG_N,), jnp.float32),
    mesh=plsc.VectorSubcoreMesh(core_axis_name="c", subcore_axis_name="s"),
    scratch_types=[
        pltpu.VMEM((3, B), jnp.int32),
        pltpu.VMEM((3, B), jnp.int32),
        pltpu.VMEM((B,), jnp.float32),
        pltpu.VMEM((DEG_PER_TILE,), jnp.float32),
        pltpu.VMEM_SHARED((DEG_N,), jnp.float32),
        pltpu.SemaphoreType.DMA,
        pltpu.SemaphoreType.DMA,
    ],
)(_sc_count_body)


def _sc_agg_body(T_hbm, pk_hbm, rec_hbm, out_hbm,
                 pk_0, pk_1, pk_2, rows_0, rows_1, rows_2,
                 rec_0, rec_1, rec_2, acc_sh,
                 spk_0, spk_1, spk_2, srow_0, srow_1, srow_2,
                 sdeg_0, sdeg_1, sdeg_2, ssc_0, ssc_1, ssc_2):
    c = lax.axis_index("c")
    s = lax.axis_index("s")

    pk = (pk_0, pk_1, pk_2)
    rows = (rows_0, rows_1, rows_2)
    rec = (rec_0, rec_1, rec_2)
    spk = (spk_0, spk_1, spk_2)
    srow = (srow_0, srow_1, srow_2)
    sdeg = (sdeg_0, sdeg_1, sdeg_2)
    ssc = (ssc_0, ssc_1, ssc_2)

    zero16 = jnp.zeros((16,), jnp.float32)

    def zrow_body(i, _):
        for c8 in range(D // 16):
            rows_0[i, pl.ds(c8 * 16, 16)] = zero16
        return 0

    lax.fori_loop(0, B, zrow_body, 0)

    for rep in range(ROWS_PER_TILE // B):
        pltpu.sync_copy(rows_0,
                        acc_sh.at[pl.ds(s * ROWS_PER_TILE + rep * B, B)])
    _tail = ROWS_PER_TILE - (ROWS_PER_TILE // B) * B
    if _tail:
        pltpu.sync_copy(
            rows_0.at[pl.ds(0, _tail)],
            acc_sh.at[pl.ds(s * ROWS_PER_TILE + ROWS_PER_TILE - _tail,
                            _tail)])
    plsc.subcore_barrier()

    w = c * N_TILES + s

    dnums = lax.GatherDimensionNumbers(
        offset_dims=(), collapsed_slice_dims=(0,), start_index_map=(0,))

    def m_start_pack(i, st):
        pltpu.async_copy(pk_hbm.at[w * MAIN_BATCHES + i], pk[st], spk[st])

    def m_wait_pack(i, st):
        pltpu.make_async_copy(pk_hbm.at[w * MAIN_BATCHES + i], pk[st],
                              spk[st]).wait()

    def m_start_gathers(st):
        pltpu.async_copy(T_hbm.at[pk[st].at[0]], rows[st], srow[st])
        pltpu.async_copy(rec_hbm.at[pk[st].at[1]], rec[st], sdeg[st])

    def m_wait_gathers(st):
        pltpu.make_async_copy(T_hbm.at[pk[st].at[0]], rows[st],
                              srow[st]).wait()
        pltpu.make_async_copy(rec_hbm.at[pk[st].at[1]], rec[st],
                              sdeg[st]).wait()

    def m_wait_scatter(st):
        pltpu.make_async_copy(rows[st], acc_sh.at[pk[st].at[2]],
                              ssc[st]).wait()

    def m_scale(st):
        rows_st, rec_st = rows[st], rec[st]

        def grp(g, _):
            r16 = rec_st[pl.ds(g * 16, 16)]
            for l in range(16):
                bc = lax.gather(r16, jnp.full((16, 1), l, jnp.int32),
                                dnums, slice_sizes=(1,),
                                mode=lax.GatherScatterMode.PROMISE_IN_BOUNDS)
                row = g * 16 + l
                for c8 in range(D // 16):
                    sl = pl.ds(c8 * 16, 16)
                    rows_st[row, sl] = rows_st[row, sl] * bc
            return 0

        lax.fori_loop(0, B // 16, grp, 0)

    def m_half(i, st, first=False, last=False):
        nxt = (st + 1) % 3
        if not first:
            m_wait_scatter(nxt)
        if not last:
            m_start_pack(i + 1, nxt)
        m_wait_gathers(st)
        if not last:
            m_wait_pack(i + 1, nxt)
            m_start_gathers(nxt)
        m_scale(st)
        pltpu.async_copy(rows[st], acc_sh.at[pk[st].at[2]], ssc[st],
                         add=True)

    m_start_pack(0, 0)
    m_wait_pack(0, 0)
    m_start_gathers(0)
    m_half(0, 0, first=True)
    m_half(1, 1, first=True)

    def m_body(j, _):
        i = 3 * j + 2
        m_half(i, 2)
        m_half(i + 1, 0)
        m_half(i + 2, 1)
        return 0

    lax.fori_loop(0, (MAIN_BATCHES - 3) // 3, m_body, 0)
    m_half(MAIN_BATCHES - 1, (MAIN_BATCHES - 1) % 3, last=True)
    m_wait_scatter((MAIN_BATCHES - 2) % 3)
    m_wait_scatter((MAIN_BATCHES - 1) % 3)
    plsc.subcore_barrier()

    pltpu.sync_copy(acc_sh.at[pl.ds(s * ROWS_PER_TILE, ROWS_PER_TILE)],
                    out_hbm.at[c, pl.ds(s * ROWS_PER_TILE, ROWS_PER_TILE)])


_sc_agg = functools.partial(
    pl.kernel,
    out_type=jax.ShapeDtypeStruct((N_SC, N_ACC, D), jnp.float32),
    mesh=plsc.VectorSubcoreMesh(core_axis_name="c", subcore_axis_name="s"),
    scratch_types=[
        pltpu.VMEM((3, B), jnp.int32),
        pltpu.VMEM((3, B), jnp.int32),
        pltpu.VMEM((3, B), jnp.int32),
        pltpu.VMEM((B, D), jnp.float32),
        pltpu.VMEM((B, D), jnp.float32),
        pltpu.VMEM((B, D), jnp.float32),
        pltpu.VMEM((B,), jnp.float32),
        pltpu.VMEM((B,), jnp.float32),
        pltpu.VMEM((B,), jnp.float32),
        pltpu.VMEM_SHARED((N_ACC, D), jnp.float32),
        pltpu.SemaphoreType.DMA,
        pltpu.SemaphoreType.DMA,
        pltpu.SemaphoreType.DMA,
        pltpu.SemaphoreType.DMA,
        pltpu.SemaphoreType.DMA,
        pltpu.SemaphoreType.DMA,
        pltpu.SemaphoreType.DMA,
        pltpu.SemaphoreType.DMA,
        pltpu.SemaphoreType.DMA,
        pltpu.SemaphoreType.DMA,
        pltpu.SemaphoreType.DMA,
        pltpu.SemaphoreType.DMA,
    ],
)(_sc_agg_body)


def kernel(feat0, feat1, edge_index, r, W0, b0, W1, b1, Wrel, brel):
    feat = jnp.concatenate([feat0, feat1], axis=0)
    bstack = jnp.stack([b0, b1])[:, None, :]

    T = pl.pallas_call(
        _transform_body,
        grid=(N_NODES // BLK,),
        in_specs=[
            pl.BlockSpec((BLK, D), lambda i: (i, 0)),
            pl.BlockSpec((D, D), lambda i: (0, 0)),
            pl.BlockSpec((D, D), lambda i: (0, 0)),
            pl.BlockSpec((2, 1, D), lambda i: (0, 0, 0)),
            pl.BlockSpec((NUM_RELS, D, D), lambda i: (0, 0, 0)),
        ],
        out_specs=pl.BlockSpec((NUM_RELS, BLK, D), lambda i: (0, i, 0)),
        out_shape=jax.ShapeDtypeStruct((NUM_RELS, N_NODES, D), jnp.float32),
    )(feat, W0, W1, bstack, Wrel)
    T_flat = T.reshape(NUM_RELS * N_NODES, D)

    u, v = edge_index[0], edge_index[1]
    key_ru = r * N_NODES + u
    key_rv = r * N_NODES + v
    npad = E_PAD - N_EDGES
    pi = jnp.arange(npad, dtype=jnp.int32)
    ru_p = jnp.concatenate([key_ru, pi % 512])
    rv_p = jnp.concatenate([key_rv, NUM_RELS * N_NODES + pi % DEG_PAD_SLOTS])
    vd_p = jnp.concatenate([v, N_NODES + pi % N_PAD_ROWS])
    pack = jnp.stack([ru_p.reshape(NB_TOTAL, B), rv_p.reshape(NB_TOTAL, B),
                      vd_p.reshape(NB_TOTAL, B)], axis=1)

    degp = _sc_count(pack)
    rec_table = pl.pallas_call(
        _recip_body,
        grid=(1,),
        in_specs=[pl.BlockSpec((N_SC, DEG_N // D, D), lambda i: (0, 0, 0))],
        out_specs=pl.BlockSpec((DEG_N // D, D), lambda i: (0, 0)),
        out_shape=jax.ShapeDtypeStruct((DEG_N // D, D), jnp.float32),
    )(degp.reshape(N_SC, DEG_N // D, D)).reshape(DEG_N)

    partial = _sc_agg(T_flat, pack, rec_table)

    out = pl.pallas_call(
        _finish_body,
        grid=(N_NODES // BLK,),
        in_specs=[
            pl.BlockSpec((N_SC, BLK, D), lambda i: (0, i, 0)),
            pl.BlockSpec((1, D), lambda i: (0, 0)),
        ],
        out_specs=pl.BlockSpec((BLK, D), lambda i: (i, 0)),
        out_shape=jax.ShapeDtypeStruct((N_NODES, D), jnp.float32),
    )(partial, brel[None, :])
    return out


# speedup vs baseline: 1.0563x; 1.0109x over previous
import functools

import jax
import jax.numpy as jnp
from jax import lax
from jax.experimental import pallas as pl
from jax.experimental.pallas import tpu as pltpu
from jax.experimental.pallas import tpu_sc as plsc

N_NODES = 10000
N_EDGES = 320000
NUM_RELS = 5
D = 128

N_TILES = 16
N_SC = 2
B = 112
MAIN_BATCHES = 93
E_PAD = N_SC * N_TILES * MAIN_BATCHES * B
NB_TOTAL = E_PAD // B
N_PAD_ROWS = 112
DEG_PAD_SLOTS = 176
N_ACC = N_NODES + N_PAD_ROWS
DEG_N = NUM_RELS * N_NODES + DEG_PAD_SLOTS
ROWS_PER_TILE = N_ACC // N_TILES
DEG_PER_TILE = DEG_N // N_TILES

BLK = 1000


def _transform_body(feat_ref, W0_ref, W1_ref, b_ref, Wrel_ref, T_ref):
    i = pl.program_id(0)
    first_half = i < (5000 // BLK)
    W = jnp.where(first_half, W0_ref[...], W1_ref[...])
    b = jnp.where(first_half, b_ref[0], b_ref[1])
    h = jnp.dot(feat_ref[...], W, preferred_element_type=jnp.float32) + b
    for rr in range(NUM_RELS):
        T_ref[rr] = jnp.dot(h, Wrel_ref[rr], preferred_element_type=jnp.float32)


def _finish_body(p_ref, brel_ref, o_ref):
    o_ref[...] = jnp.maximum(p_ref[0] + p_ref[1] + brel_ref[...], 0.0)


def _recip_body(p_ref, o_ref):
    d = p_ref[0] + p_ref[1]
    o_ref[...] = jnp.where(d > 0, 1.0 / jnp.maximum(d, 1.0), 0.0)


def _sc_count_body(pk_hbm, degp_hbm,
                   pk_a, pk_b, pk_c, ones_v, zdeg_v, deg_sh,
                   spk_a, spk_b, spk_c, ssc_a, ssc_b, ssc_c):
    c = lax.axis_index("c")
    s = lax.axis_index("s")
    w = c * N_TILES + s

    pk = (pk_a, pk_b, pk_c)
    spk = (spk_a, spk_b, spk_c)
    ssc = (ssc_a, ssc_b, ssc_c)

    one16 = jnp.ones((16,), jnp.float32)
    zero16 = jnp.zeros((16,), jnp.float32)
    for g in range(B // 16):
        ones_v[pl.ds(g * 16, 16)] = one16

    def zdeg_body(i, _):
        zdeg_v[pl.ds(i * 16, 16)] = zero16
        return 0

    lax.fori_loop(0, DEG_PER_TILE // 16, zdeg_body, 0)
    pltpu.sync_copy(zdeg_v, deg_sh.at[pl.ds(s * DEG_PER_TILE, DEG_PER_TILE)])
    plsc.subcore_barrier()

    def cnt_start_pack(k, st):
        pltpu.async_copy(pk_hbm.at[w * MAIN_BATCHES + k], pk[st], spk[st])

    def cnt_wait_pack(k, st):
        pltpu.make_async_copy(pk_hbm.at[w * MAIN_BATCHES + k], pk[st],
                              spk[st]).wait()

    def cnt_wait_scatter(st):
        pltpu.make_async_copy(ones_v, deg_sh.at[pk[st].at[1]],
                              ssc[st]).wait()

    def cnt_half(k, st, first=False, last=False):
        nxt = (st + 1) % 3
        if not first:
            cnt_wait_scatter(nxt)
        if not last:
            cnt_start_pack(k + 1, nxt)
        cnt_wait_pack(k, st)
        pltpu.async_copy(ones_v, deg_sh.at[pk[st].at[1]], ssc[st], add=True)

    cnt_start_pack(0, 0)
    cnt_half(0, 0, first=True)
    cnt_half(1, 1, first=True)

    def cnt_body(j, _):
        k = 3 * j + 2
        cnt_half(k, 2)
        cnt_half(k + 1, 0)
        cnt_half(k + 2, 1)
        return 0

    lax.fori_loop(0, (MAIN_BATCHES - 3) // 3, cnt_body, 0)
    cnt_half(MAIN_BATCHES - 1, (MAIN_BATCHES - 1) % 3, last=True)
    cnt_wait_scatter((MAIN_BATCHES - 2) % 3)
    cnt_wait_scatter((MAIN_BATCHES - 1) % 3)
    plsc.subcore_barrier()

    pltpu.sync_copy(deg_sh.at[pl.ds(s * DEG_PER_TILE, DEG_PER_TILE)], zdeg_v)
    pltpu.sync_copy(zdeg_v,
                    degp_hbm.at[pl.ds(c * DEG_N + s * DEG_PER_TILE,
                                      DEG_PER_TILE)])


_sc_count = functools.partial(
    pl.kernel,
    out_type=jax.ShapeDtypeStruct((N_SC * DEG_N,), jnp.float32),
    mesh=plsc.VectorSubcoreMesh(core_axis_name="c", subcore_axis_name="s"),
    scratch_types=[
        pltpu.VMEM((3, B), jnp.int32),
        pltpu.VMEM((3, B), jnp.int32),
        pltpu.VMEM((3, B), jnp.int32),
        pltpu.VMEM((B,), jnp.float32),
        pltpu.VMEM((DEG_PER_TILE,), jnp.float32),
        pltpu.VMEM_SHARED((DEG_N,), jnp.float32),
        pltpu.SemaphoreType.DMA,
        pltpu.SemaphoreType.DMA,
        pltpu.SemaphoreType.DMA,
        pltpu.SemaphoreType.DMA,
        pltpu.SemaphoreType.DMA,
        pltpu.SemaphoreType.DMA,
    ],
)(_sc_count_body)


def _sc_agg_body(T_hbm, pk_hbm, rec_hbm, out_hbm,
                 pk_0, pk_1, pk_2, rows_0, rows_1, rows_2,
                 rec_0, rec_1, rec_2, acc_sh,
                 spk_0, spk_1, spk_2, srow_0, srow_1, srow_2,
                 sdeg_0, sdeg_1, sdeg_2, ssc_0, ssc_1, ssc_2):
    c = lax.axis_index("c")
    s = lax.axis_index("s")

    pk = (pk_0, pk_1, pk_2)
    rows = (rows_0, rows_1, rows_2)
    rec = (rec_0, rec_1, rec_2)
    spk = (spk_0, spk_1, spk_2)
    srow = (srow_0, srow_1, srow_2)
    sdeg = (sdeg_0, sdeg_1, sdeg_2)
    ssc = (ssc_0, ssc_1, ssc_2)

    zero16 = jnp.zeros((16,), jnp.float32)

    def zrow_body(i, _):
        for c8 in range(D // 16):
            rows_0[i, pl.ds(c8 * 16, 16)] = zero16
        return 0

    lax.fori_loop(0, B, zrow_body, 0)

    for rep in range(ROWS_PER_TILE // B):
        pltpu.sync_copy(rows_0,
                        acc_sh.at[pl.ds(s * ROWS_PER_TILE + rep * B, B)])
    _tail = ROWS_PER_TILE - (ROWS_PER_TILE // B) * B
    if _tail:
        pltpu.sync_copy(
            rows_0.at[pl.ds(0, _tail)],
            acc_sh.at[pl.ds(s * ROWS_PER_TILE + ROWS_PER_TILE - _tail,
                            _tail)])
    plsc.subcore_barrier()

    w = c * N_TILES + s

    dnums = lax.GatherDimensionNumbers(
        offset_dims=(), collapsed_slice_dims=(0,), start_index_map=(0,))

    def m_start_pack(i, st):
        pltpu.async_copy(pk_hbm.at[w * MAIN_BATCHES + i], pk[st], spk[st])

    def m_wait_pack(i, st):
        pltpu.make_async_copy(pk_hbm.at[w * MAIN_BATCHES + i], pk[st],
                              spk[st]).wait()

    def m_start_gathers(st):
        pltpu.async_copy(T_hbm.at[pk[st].at[0]], rows[st], srow[st])
        pltpu.async_copy(rec_hbm.at[pk[st].at[1]], rec[st], sdeg[st])

    def m_wait_gathers(st):
        pltpu.make_async_copy(T_hbm.at[pk[st].at[0]], rows[st],
                              srow[st]).wait()
        pltpu.make_async_copy(rec_hbm.at[pk[st].at[1]], rec[st],
                              sdeg[st]).wait()

    def m_wait_scatter(st):
        pltpu.make_async_copy(rows[st], acc_sh.at[pk[st].at[2]],
                              ssc[st]).wait()

    def m_scale(st):
        rows_st, rec_st = rows[st], rec[st]

        def grp(g, _):
            r16 = rec_st[pl.ds(g * 16, 16)]
            for l in range(16):
                bc = lax.gather(r16, jnp.full((16, 1), l, jnp.int32),
                                dnums, slice_sizes=(1,),
                                mode=lax.GatherScatterMode.PROMISE_IN_BOUNDS)
                row = g * 16 + l
                for c8 in range(D // 16):
                    sl = pl.ds(c8 * 16, 16)
                    rows_st[row, sl] = rows_st[row, sl] * bc
            return 0

        lax.fori_loop(0, B // 16, grp, 0)

    def m_half(i, st, first=False, last=False):
        nxt = (st + 1) % 3
        if not first:
            m_wait_scatter(nxt)
        if not last:
            m_start_pack(i + 1, nxt)
        m_wait_gathers(st)
        if not last:
            m_wait_pack(i + 1, nxt)
            m_start_gathers(nxt)
        m_scale(st)
        pltpu.async_copy(rows[st], acc_sh.at[pk[st].at[2]], ssc[st],
                         add=True)

    m_start_pack(0, 0)
    m_wait_pack(0, 0)
    m_start_gathers(0)
    m_half(0, 0, first=True)
    m_half(1, 1, first=True)

    def m_body(j, _):
        i = 3 * j + 2
        m_half(i, 2)
        m_half(i + 1, 0)
        m_half(i + 2, 1)
        return 0

    lax.fori_loop(0, (MAIN_BATCHES - 3) // 3, m_body, 0)
    m_half(MAIN_BATCHES - 1, (MAIN_BATCHES - 1) % 3, last=True)
    m_wait_scatter((MAIN_BATCHES - 2) % 3)
    m_wait_scatter((MAIN_BATCHES - 1) % 3)
    plsc.subcore_barrier()

    pltpu.sync_copy(acc_sh.at[pl.ds(s * ROWS_PER_TILE, ROWS_PER_TILE)],
                    out_hbm.at[c, pl.ds(s * ROWS_PER_TILE, ROWS_PER_TILE)])


_sc_agg = functools.partial(
    pl.kernel,
    out_type=jax.ShapeDtypeStruct((N_SC, N_ACC, D), jnp.float32),
    mesh=plsc.VectorSubcoreMesh(core_axis_name="c", subcore_axis_name="s"),
    scratch_types=[
        pltpu.VMEM((3, B), jnp.int32),
        pltpu.VMEM((3, B), jnp.int32),
        pltpu.VMEM((3, B), jnp.int32),
        pltpu.VMEM((B, D), jnp.float32),
        pltpu.VMEM((B, D), jnp.float32),
        pltpu.VMEM((B, D), jnp.float32),
        pltpu.VMEM((B,), jnp.float32),
        pltpu.VMEM((B,), jnp.float32),
        pltpu.VMEM((B,), jnp.float32),
        pltpu.VMEM_SHARED((N_ACC, D), jnp.float32),
        pltpu.SemaphoreType.DMA,
        pltpu.SemaphoreType.DMA,
        pltpu.SemaphoreType.DMA,
        pltpu.SemaphoreType.DMA,
        pltpu.SemaphoreType.DMA,
        pltpu.SemaphoreType.DMA,
        pltpu.SemaphoreType.DMA,
        pltpu.SemaphoreType.DMA,
        pltpu.SemaphoreType.DMA,
        pltpu.SemaphoreType.DMA,
        pltpu.SemaphoreType.DMA,
        pltpu.SemaphoreType.DMA,
    ],
)(_sc_agg_body)


def kernel(feat0, feat1, edge_index, r, W0, b0, W1, b1, Wrel, brel):
    feat = jnp.concatenate([feat0, feat1], axis=0)
    bstack = jnp.stack([b0, b1])[:, None, :]

    T = pl.pallas_call(
        _transform_body,
        grid=(N_NODES // BLK,),
        in_specs=[
            pl.BlockSpec((BLK, D), lambda i: (i, 0)),
            pl.BlockSpec((D, D), lambda i: (0, 0)),
            pl.BlockSpec((D, D), lambda i: (0, 0)),
            pl.BlockSpec((2, 1, D), lambda i: (0, 0, 0)),
            pl.BlockSpec((NUM_RELS, D, D), lambda i: (0, 0, 0)),
        ],
        out_specs=pl.BlockSpec((NUM_RELS, BLK, D), lambda i: (0, i, 0)),
        out_shape=jax.ShapeDtypeStruct((NUM_RELS, N_NODES, D), jnp.float32),
    )(feat, W0, W1, bstack, Wrel)
    T_flat = T.reshape(NUM_RELS * N_NODES, D)

    u, v = edge_index[0], edge_index[1]
    key_ru = r * N_NODES + u
    key_rv = r * N_NODES + v
    npad = E_PAD - N_EDGES
    pi = jnp.arange(npad, dtype=jnp.int32)
    ru_p = jnp.concatenate([key_ru, pi % 512])
    rv_p = jnp.concatenate([key_rv, NUM_RELS * N_NODES + pi % DEG_PAD_SLOTS])
    vd_p = jnp.concatenate([v, N_NODES + pi % N_PAD_ROWS])
    pack = jnp.stack([ru_p.reshape(NB_TOTAL, B), rv_p.reshape(NB_TOTAL, B),
                      vd_p.reshape(NB_TOTAL, B)], axis=1)

    degp = _sc_count(pack)
    rec_table = pl.pallas_call(
        _recip_body,
        grid=(1,),
        in_specs=[pl.BlockSpec((N_SC, DEG_N // D, D), lambda i: (0, 0, 0))],
        out_specs=pl.BlockSpec((DEG_N // D, D), lambda i: (0, 0)),
        out_shape=jax.ShapeDtypeStruct((DEG_N // D, D), jnp.float32),
    )(degp.reshape(N_SC, DEG_N // D, D)).reshape(DEG_N)

    partial = _sc_agg(T_flat, pack, rec_table)

    out = pl.pallas_call(
        _finish_body,
        grid=(N_NODES // BLK,),
        in_specs=[
            pl.BlockSpec((N_SC, BLK, D), lambda i: (0, i, 0)),
            pl.BlockSpec((1, D), lambda i: (0, 0)),
        ],
        out_specs=pl.BlockSpec((BLK, D), lambda i: (i, 0)),
        out_shape=jax.ShapeDtypeStruct((N_NODES, D), jnp.float32),
    )(partial, brel[None, :])
    return out
